# baseline (speedup 1.0000x reference)
import jax
import jax.numpy as jnp
from jax import lax
from jax.experimental import pallas as pl
from jax.experimental.pallas import tpu as pltpu
from jax.experimental.pallas import tpu_sc as plsc

_N = 12


def _compact_body(in_hbm, out_hbm, in_s, out_s):
    pltpu.sync_copy(in_hbm, in_s)

    cnt = jnp.int32(0)
    plan = []
    for i in range(_N):
        ok = in_s[i] < 1.0
        plan.append((i, cnt))
        cnt = cnt + jnp.where(ok, 1, 0)

    for i, pos in plan:
        out_s[pos] = in_s[_N + i]

    st0 = in_s[_N]

    def pad_cond(j):
        return j < _N

    def pad_body(j):
        out_s[j] = st0
        return j + 1

    lax.while_loop(pad_cond, pad_body, cnt)

    pltpu.sync_copy(out_s, out_hbm)


def kernel(t2, stored_tensor):
    packed = jnp.concatenate([t2.reshape(-1), stored_tensor.reshape(-1)])
    mesh = plsc.ScalarSubcoreMesh(axis_name="c", num_cores=1)
    run = pl.kernel(
        _compact_body,
        mesh=mesh,
        out_type=jax.ShapeDtypeStruct((_N,), jnp.float32),
        scratch_types=[
            pltpu.SMEM((2 * _N,), jnp.float32),
            pltpu.SMEM((_N,), jnp.float32),
        ],
        compiler_params=pltpu.CompilerParams(needs_layout_passes=False),
    )
    return run(packed)
